# baseline (speedup 1.0000x reference)
import functools

import jax
import jax.numpy as jnp
from jax import lax
from jax.experimental import pallas as pl
from jax.experimental.pallas import tpu as pltpu
from jax.experimental.pallas import tpu_sc as plsc

N_NODES = 10000
N_PAD = 10240
NW = 32
K = 128
ROWS_PER_TILE = N_PAD // 16
ZCH = ROWS_PER_TILE // K


def _make_sc_aggregate(n_chunks: int, d: int):
    mesh = plsc.VectorSubcoreMesh(
        core_axis_name="c", subcore_axis_name="s", num_cores=2, num_subcores=16)

    @functools.partial(
        pl.kernel,
        mesh=mesh,
        out_type=jax.ShapeDtypeStruct((2, N_PAD, d), jnp.float32),
        scratch_types=[
            pltpu.VMEM((n_chunks, K), jnp.int32),
            pltpu.VMEM((n_chunks, K), jnp.int32),
            [pltpu.VMEM((K, d), jnp.float32)] * 4,
            pltpu.VMEM((K, d), jnp.float32),
            pltpu.VMEM_SHARED((N_PAD, d), jnp.float32),
            [pltpu.SemaphoreType.DMA] * 4,
            [pltpu.SemaphoreType.DMA] * 4,
            pltpu.SemaphoreType.DMA,
        ],
        compiler_params=pltpu.CompilerParams(use_tc_tiling_on_sc=False),
    )
    def agg_kernel(h_hbm, src_hbm, dst_hbm, out_hbm,
                   src_v, dst_v, rows, zbuf, acc, gsem, ssem, isem):
        c = lax.axis_index("c")
        s = lax.axis_index("s")
        wid = c * 16 + s
        nb = 4

        pltpu.async_copy(src_hbm.at[wid], src_v, isem)
        pltpu.async_copy(dst_hbm.at[wid], dst_v, isem)

        zv = jnp.zeros((16,), jnp.float32)

        def zrow(i, _):
            for j in range(d // 16):
                zbuf[i, pl.ds(j * 16, 16)] = zv
            return 0

        lax.fori_loop(0, K, zrow, 0)
        for z in range(ZCH):
            pltpu.sync_copy(zbuf, acc.at[pl.ds(s * ROWS_PER_TILE + z * K, K)])
        plsc.subcore_barrier()

        pltpu.make_async_copy(src_hbm.at[wid], src_v, isem).wait()
        pltpu.make_async_copy(dst_hbm.at[wid], dst_v, isem).wait()

        def step(j, _):
            pltpu.async_copy(h_hbm.at[src_v.at[j]], rows[0], gsem[0]).wait()
            pltpu.sync_copy(rows[0], acc.at[dst_v.at[j]], add=True)
            return 0

        lax.fori_loop(0, n_chunks, step, 0)

        plsc.subcore_barrier()
        pltpu.sync_copy(
            acc.at[pl.ds(s * ROWS_PER_TILE, ROWS_PER_TILE)],
            out_hbm.at[c, pl.ds(s * ROWS_PER_TILE, ROWS_PER_TILE)],
        )

    return agg_kernel


def _mlp_block(h_ref, a0_ref, a1_ref, w1_ref, b1_ref, w2_ref, b2_ref, o_ref):
    z = h_ref[...] + a0_ref[...] + a1_ref[...]
    y = jnp.maximum(
        jnp.dot(z, w1_ref[...], preferred_element_type=jnp.float32,
                precision=lax.Precision.HIGHEST) + b1_ref[...], 0.0)
    o = jnp.dot(y, w2_ref[...], preferred_element_type=jnp.float32,
                precision=lax.Precision.HIGHEST) + b2_ref[...]
    o_ref[...] = jnp.maximum(o, 0.0)


def _mlp_apply(h, a0, a1, w1, b1, w2, b2):
    n, d = h.shape
    out_c = w2.shape[1]
    bn = 512
    grid = (pl.cdiv(n, bn),)
    return pl.pallas_call(
        _mlp_block,
        grid=grid,
        in_specs=[
            pl.BlockSpec((bn, d), lambda i: (i, 0)),
            pl.BlockSpec((bn, d), lambda i: (i, 0)),
            pl.BlockSpec((bn, d), lambda i: (i, 0)),
            pl.BlockSpec((d, w1.shape[1]), lambda i: (0, 0)),
            pl.BlockSpec((1, w1.shape[1]), lambda i: (0, 0)),
            pl.BlockSpec((w1.shape[1], out_c), lambda i: (0, 0)),
            pl.BlockSpec((1, out_c), lambda i: (0, 0)),
        ],
        out_specs=pl.BlockSpec((bn, out_c), lambda i: (i, 0)),
        out_shape=jax.ShapeDtypeStruct((n, out_c), jnp.float32),
    )(h, a0, a1, w1, b1, w2, b2)


def kernel(x, edge_index, params):
    n, d_in = x.shape
    e = edge_index.shape[1]
    epw = pl.cdiv(e, NW * 4 * K) * 4 * K
    n_chunks = epw // K
    e_pad = NW * epw

    src = edge_index[0]
    dst = edge_index[1]
    src_p = jnp.concatenate(
        [src, jnp.zeros((e_pad - e,), jnp.int32)]).reshape(NW, n_chunks, K)
    dst_p = jnp.concatenate(
        [dst, jnp.full((e_pad - e,), n, jnp.int32)]).reshape(NW, n_chunks, K)

    h = x
    for (w1, b1, w2, b2) in params:
        d = h.shape[1]
        dc = 64 if d > 64 else d
        parts = []
        for c0 in range(0, d, dc):
            hcol = h[:, c0:c0 + dc] if d > dc else h
            parts.append(_make_sc_aggregate(n_chunks, dc)(hcol, src_p, dst_p))
        agg = jnp.concatenate(parts, axis=-1) if len(parts) > 1 else parts[0]
        h = _mlp_apply(h, agg[0, :n], agg[1, :n],
                       w1, b1.reshape(1, -1), w2, b2.reshape(1, -1))
    return h


# speedup vs baseline: 2.0086x; 2.0086x over previous
import functools

import jax
import jax.numpy as jnp
from jax import lax
from jax.experimental import pallas as pl
from jax.experimental.pallas import tpu as pltpu
from jax.experimental.pallas import tpu_sc as plsc

N_NODES = 10000
N_PAD = 10240
NW = 32
K = 128
ROWS_PER_TILE = N_PAD // 16
ZCH = ROWS_PER_TILE // K


def _make_sc_aggregate(n_chunks: int, d: int):
    mesh = plsc.VectorSubcoreMesh(
        core_axis_name="c", subcore_axis_name="s", num_cores=2, num_subcores=16)

    @functools.partial(
        pl.kernel,
        mesh=mesh,
        out_type=jax.ShapeDtypeStruct((2, N_PAD, d), jnp.float32),
        scratch_types=[
            pltpu.VMEM((n_chunks, K), jnp.int32),
            pltpu.VMEM((n_chunks, K), jnp.int32),
            pltpu.VMEM((K, d), jnp.float32),
            pltpu.VMEM((K, d), jnp.float32),
            pltpu.VMEM_SHARED((N_PAD, d), jnp.float32),
            pltpu.SemaphoreType.DMA,
        ],
        compiler_params=pltpu.CompilerParams(use_tc_tiling_on_sc=False),
    )
    def agg_kernel(h_hbm, src_hbm, dst_hbm, out_hbm,
                   src_v, dst_v, rows0, zbuf, acc, sem0):
        c = lax.axis_index("c")
        s = lax.axis_index("s")
        wid = c * 16 + s

        zv = jnp.zeros((16,), jnp.float32)

        def zrow(i, _):
            for j in range(d // 16):
                zbuf[i, pl.ds(j * 16, 16)] = zv
            return 0

        lax.fori_loop(0, K, zrow, 0)
        for z in range(ZCH):
            pltpu.sync_copy(zbuf, acc.at[pl.ds(s * ROWS_PER_TILE + z * K, K)])
        plsc.subcore_barrier()

        pltpu.sync_copy(src_hbm.at[wid], src_v)
        pltpu.sync_copy(dst_hbm.at[wid], dst_v)

        def step(j, _):
            pltpu.async_copy(h_hbm.at[src_v.at[j]], rows0, sem0).wait()
            pltpu.sync_copy(rows0, acc.at[dst_v.at[j]], add=True)
            return 0

        lax.fori_loop(0, n_chunks, step, 0)

        plsc.subcore_barrier()
        pltpu.sync_copy(
            acc.at[pl.ds(s * ROWS_PER_TILE, ROWS_PER_TILE)],
            out_hbm.at[c, pl.ds(s * ROWS_PER_TILE, ROWS_PER_TILE)],
        )

    return agg_kernel


def _mlp_block(h_ref, a0_ref, a1_ref, w1_ref, b1_ref, w2_ref, b2_ref, o_ref):
    z = h_ref[...] + a0_ref[...] + a1_ref[...]
    y = jnp.maximum(
        jnp.dot(z, w1_ref[...], preferred_element_type=jnp.float32,
                precision=lax.Precision.HIGHEST) + b1_ref[...], 0.0)
    o = jnp.dot(y, w2_ref[...], preferred_element_type=jnp.float32,
                precision=lax.Precision.HIGHEST) + b2_ref[...]
    o_ref[...] = jnp.maximum(o, 0.0)


def _mlp_apply(h, a0, a1, w1, b1, w2, b2):
    n, d = h.shape
    out_c = w2.shape[1]
    bn = 512
    grid = (pl.cdiv(n, bn),)
    return pl.pallas_call(
        _mlp_block,
        grid=grid,
        in_specs=[
            pl.BlockSpec((bn, d), lambda i: (i, 0)),
            pl.BlockSpec((bn, d), lambda i: (i, 0)),
            pl.BlockSpec((bn, d), lambda i: (i, 0)),
            pl.BlockSpec((d, w1.shape[1]), lambda i: (0, 0)),
            pl.BlockSpec((1, w1.shape[1]), lambda i: (0, 0)),
            pl.BlockSpec((w1.shape[1], out_c), lambda i: (0, 0)),
            pl.BlockSpec((1, out_c), lambda i: (0, 0)),
        ],
        out_specs=pl.BlockSpec((bn, out_c), lambda i: (i, 0)),
        out_shape=jax.ShapeDtypeStruct((n, out_c), jnp.float32),
    )(h, a0, a1, w1, b1, w2, b2)


def kernel(x, edge_index, params):
    n, d_in = x.shape
    e = edge_index.shape[1]
    epw = pl.cdiv(e, NW * 4 * K) * 4 * K
    n_chunks = epw // K
    e_pad = NW * epw

    src = edge_index[0]
    dst = edge_index[1]
    npad_e = e_pad - e
    pad_iota = jnp.arange(npad_e, dtype=jnp.int32)
    src_p = jnp.concatenate(
        [src, pad_iota % n]).reshape(NW, n_chunks, K)
    dst_p = jnp.concatenate(
        [dst, n + pad_iota % (N_PAD - n)]).reshape(NW, n_chunks, K)

    h = x
    for (w1, b1, w2, b2) in params:
        d = h.shape[1]
        dc = 64 if d > 64 else d
        parts = []
        for c0 in range(0, d, dc):
            hcol = h[:, c0:c0 + dc] if d > dc else h
            parts.append(_make_sc_aggregate(n_chunks, dc)(hcol, src_p, dst_p))
        agg = jnp.concatenate(parts, axis=-1) if len(parts) > 1 else parts[0]
        h = _mlp_apply(h, agg[0, :n], agg[1, :n],
                       w1, b1.reshape(1, -1), w2, b2.reshape(1, -1))
    return h


# speedup vs baseline: 2.8751x; 1.4314x over previous
import functools

import jax
import jax.numpy as jnp
from jax import lax
from jax.experimental import pallas as pl
from jax.experimental.pallas import tpu as pltpu
from jax.experimental.pallas import tpu_sc as plsc

N_NODES = 10000
N_PAD = 10240
NW = 32
K = 128
ROWS_PER_TILE = N_PAD // 16
ZCH = ROWS_PER_TILE // K


def _make_sc_aggregate(n_chunks: int, d: int):
    mesh = plsc.VectorSubcoreMesh(
        core_axis_name="c", subcore_axis_name="s", num_cores=2, num_subcores=16)

    @functools.partial(
        pl.kernel,
        mesh=mesh,
        out_type=jax.ShapeDtypeStruct((2, N_PAD, d), jnp.float32),
        scratch_types=[
            pltpu.VMEM((n_chunks, K), jnp.int32),
            pltpu.VMEM((n_chunks, K), jnp.int32),
            pltpu.VMEM((K, d), jnp.float32),
            pltpu.VMEM((K, d), jnp.float32),
            pltpu.VMEM((K, d), jnp.float32),
            pltpu.VMEM_SHARED((N_PAD, d), jnp.float32),
            pltpu.SemaphoreType.DMA,
            pltpu.SemaphoreType.DMA,
        ],
        compiler_params=pltpu.CompilerParams(use_tc_tiling_on_sc=False),
    )
    def agg_kernel(h_hbm, src_hbm, dst_hbm, out_hbm,
                   src_v, dst_v, rows0, rows1, zbuf, acc, sem0, sem1):
        c = lax.axis_index("c")
        s = lax.axis_index("s")
        wid = c * 16 + s

        zv = jnp.zeros((16,), jnp.float32)

        def zrow(i, _):
            for j in range(d // 16):
                zbuf[i, pl.ds(j * 16, 16)] = zv
            return 0

        lax.fori_loop(0, K, zrow, 0)
        for z in range(ZCH):
            pltpu.sync_copy(zbuf, acc.at[pl.ds(s * ROWS_PER_TILE + z * K, K)])
        plsc.subcore_barrier()

        pltpu.sync_copy(src_hbm.at[wid], src_v)
        pltpu.sync_copy(dst_hbm.at[wid], dst_v)

        pltpu.async_copy(h_hbm.at[src_v.at[0]], rows0, sem0)

        def step(j, _):
            pltpu.async_copy(h_hbm.at[src_v.at[2 * j + 1]], rows1, sem1)
            pltpu.make_async_copy(h_hbm.at[src_v.at[2 * j]], rows0, sem0).wait()
            pltpu.sync_copy(rows0, acc.at[dst_v.at[2 * j]], add=True)
            pltpu.async_copy(h_hbm.at[src_v.at[2 * j + 2]], rows0, sem0)
            pltpu.make_async_copy(h_hbm.at[src_v.at[2 * j + 1]], rows1, sem1).wait()
            pltpu.sync_copy(rows1, acc.at[dst_v.at[2 * j + 1]], add=True)
            return 0

        lax.fori_loop(0, n_chunks // 2 - 1, step, 0)

        t0 = n_chunks - 2
        pltpu.async_copy(h_hbm.at[src_v.at[t0 + 1]], rows1, sem1)
        pltpu.make_async_copy(h_hbm.at[src_v.at[t0]], rows0, sem0).wait()
        pltpu.sync_copy(rows0, acc.at[dst_v.at[t0]], add=True)
        pltpu.make_async_copy(h_hbm.at[src_v.at[t0 + 1]], rows1, sem1).wait()
        pltpu.sync_copy(rows1, acc.at[dst_v.at[t0 + 1]], add=True)

        plsc.subcore_barrier()
        pltpu.sync_copy(
            acc.at[pl.ds(s * ROWS_PER_TILE, ROWS_PER_TILE)],
            out_hbm.at[c, pl.ds(s * ROWS_PER_TILE, ROWS_PER_TILE)],
        )

    return agg_kernel


def _mlp_block(h_ref, a0_ref, a1_ref, w1_ref, b1_ref, w2_ref, b2_ref, o_ref):
    z = h_ref[...] + a0_ref[...] + a1_ref[...]
    y = jnp.maximum(
        jnp.dot(z, w1_ref[...], preferred_element_type=jnp.float32,
                precision=lax.Precision.HIGHEST) + b1_ref[...], 0.0)
    o = jnp.dot(y, w2_ref[...], preferred_element_type=jnp.float32,
                precision=lax.Precision.HIGHEST) + b2_ref[...]
    o_ref[...] = jnp.maximum(o, 0.0)


def _mlp_apply(h, a0, a1, w1, b1, w2, b2):
    n, d = h.shape
    out_c = w2.shape[1]
    bn = 512
    grid = (pl.cdiv(n, bn),)
    return pl.pallas_call(
        _mlp_block,
        grid=grid,
        in_specs=[
            pl.BlockSpec((bn, d), lambda i: (i, 0)),
            pl.BlockSpec((bn, d), lambda i: (i, 0)),
            pl.BlockSpec((bn, d), lambda i: (i, 0)),
            pl.BlockSpec((d, w1.shape[1]), lambda i: (0, 0)),
            pl.BlockSpec((1, w1.shape[1]), lambda i: (0, 0)),
            pl.BlockSpec((w1.shape[1], out_c), lambda i: (0, 0)),
            pl.BlockSpec((1, out_c), lambda i: (0, 0)),
        ],
        out_specs=pl.BlockSpec((bn, out_c), lambda i: (i, 0)),
        out_shape=jax.ShapeDtypeStruct((n, out_c), jnp.float32),
    )(h, a0, a1, w1, b1, w2, b2)


def kernel(x, edge_index, params):
    n, d_in = x.shape
    e = edge_index.shape[1]
    epw = pl.cdiv(e, NW * 4 * K) * 4 * K
    n_chunks = epw // K
    e_pad = NW * epw

    src = edge_index[0]
    dst = edge_index[1]
    npad_e = e_pad - e
    pad_iota = jnp.arange(npad_e, dtype=jnp.int32)
    src_p = jnp.concatenate(
        [src, pad_iota % n]).reshape(NW, n_chunks, K)
    dst_p = jnp.concatenate(
        [dst, n + pad_iota % (N_PAD - n)]).reshape(NW, n_chunks, K)

    h = x
    for (w1, b1, w2, b2) in params:
        d = h.shape[1]
        dc = 64 if d > 64 else d
        parts = []
        for c0 in range(0, d, dc):
            hcol = h[:, c0:c0 + dc] if d > dc else h
            parts.append(_make_sc_aggregate(n_chunks, dc)(hcol, src_p, dst_p))
        agg = jnp.concatenate(parts, axis=-1) if len(parts) > 1 else parts[0]
        h = _mlp_apply(h, agg[0, :n], agg[1, :n],
                       w1, b1.reshape(1, -1), w2, b2.reshape(1, -1))
    return h


# speedup vs baseline: 3.0442x; 1.0588x over previous
import functools

import jax
import jax.numpy as jnp
from jax import lax
from jax.experimental import pallas as pl
from jax.experimental.pallas import tpu as pltpu
from jax.experimental.pallas import tpu_sc as plsc

N_NODES = 10000
N_PAD = 10240
NW = 32
K = 128
ROWS_PER_TILE = N_PAD // 16
ZCH = ROWS_PER_TILE // K


def _make_sc_aggregate(n_chunks: int, d: int):
    mesh = plsc.VectorSubcoreMesh(
        core_axis_name="c", subcore_axis_name="s", num_cores=2, num_subcores=16)

    @functools.partial(
        pl.kernel,
        mesh=mesh,
        out_type=jax.ShapeDtypeStruct((2, N_PAD, d), jnp.float32),
        scratch_types=[
            pltpu.VMEM((n_chunks, K), jnp.int32),
            pltpu.VMEM((n_chunks, K), jnp.int32),
            pltpu.VMEM((K, d), jnp.float32),
            pltpu.VMEM((K, d), jnp.float32),
            pltpu.VMEM((K, d), jnp.float32),
            pltpu.VMEM_SHARED((N_PAD, d), jnp.float32),
            pltpu.SemaphoreType.DMA,
            pltpu.SemaphoreType.DMA,
        ],
        compiler_params=pltpu.CompilerParams(use_tc_tiling_on_sc=False),
    )
    def agg_kernel(h_hbm, src_hbm, dst_hbm, out_hbm,
                   src_v, dst_v, rows0, rows1, zbuf, acc, sem0, sem1):
        c = lax.axis_index("c")
        s = lax.axis_index("s")
        wid = c * 16 + s

        zv = jnp.zeros((16,), jnp.float32)

        def zrow(i, _):
            for j in range(d // 16):
                zbuf[i, pl.ds(j * 16, 16)] = zv
            return 0

        lax.fori_loop(0, K, zrow, 0)
        for z in range(ZCH):
            pltpu.sync_copy(zbuf, acc.at[pl.ds(s * ROWS_PER_TILE + z * K, K)])
        plsc.subcore_barrier()

        pltpu.sync_copy(src_hbm.at[wid], src_v)
        pltpu.sync_copy(dst_hbm.at[wid], dst_v)

        pltpu.async_copy(h_hbm.at[src_v.at[0]], rows0, sem0)

        def step(j, _):
            pltpu.async_copy(h_hbm.at[src_v.at[2 * j + 1]], rows1, sem1)
            pltpu.make_async_copy(h_hbm.at[src_v.at[2 * j]], rows0, sem0).wait()
            pltpu.sync_copy(rows0, acc.at[dst_v.at[2 * j]], add=True)
            pltpu.async_copy(h_hbm.at[src_v.at[2 * j + 2]], rows0, sem0)
            pltpu.make_async_copy(h_hbm.at[src_v.at[2 * j + 1]], rows1, sem1).wait()
            pltpu.sync_copy(rows1, acc.at[dst_v.at[2 * j + 1]], add=True)
            return 0

        lax.fori_loop(0, n_chunks // 2 - 1, step, 0)

        t0 = n_chunks - 2
        pltpu.async_copy(h_hbm.at[src_v.at[t0 + 1]], rows1, sem1)
        pltpu.make_async_copy(h_hbm.at[src_v.at[t0]], rows0, sem0).wait()
        pltpu.sync_copy(rows0, acc.at[dst_v.at[t0]], add=True)
        pltpu.make_async_copy(h_hbm.at[src_v.at[t0 + 1]], rows1, sem1).wait()
        pltpu.sync_copy(rows1, acc.at[dst_v.at[t0 + 1]], add=True)

        plsc.subcore_barrier()
        pltpu.sync_copy(
            acc.at[pl.ds(s * ROWS_PER_TILE, ROWS_PER_TILE)],
            out_hbm.at[c, pl.ds(s * ROWS_PER_TILE, ROWS_PER_TILE)],
        )

    return agg_kernel


def _mlp_apply(h, aggs, w1, b1, w2, b2):
    n, d = h.shape
    out_c = w2.shape[1]
    bn = 512
    grid = (pl.cdiv(n, bn),)
    n_parts = len(aggs)

    def body(h_ref, *refs):
        agg_refs = refs[:n_parts]
        w1_ref, b1_ref, w2_ref, b2_ref, o_ref = refs[n_parts:]
        parts = [ar[0] + ar[1] for ar in agg_refs]
        agg = jnp.concatenate(parts, axis=-1) if n_parts > 1 else parts[0]
        z = h_ref[...] + agg
        y = jnp.maximum(
            jnp.dot(z, w1_ref[...], preferred_element_type=jnp.float32,
                    precision=lax.Precision.HIGHEST) + b1_ref[...], 0.0)
        o = jnp.dot(y, w2_ref[...], preferred_element_type=jnp.float32,
                    precision=lax.Precision.HIGHEST) + b2_ref[...]
        o_ref[...] = jnp.maximum(o, 0.0)

    return pl.pallas_call(
        body,
        grid=grid,
        in_specs=[pl.BlockSpec((bn, d), lambda i: (i, 0))]
        + [pl.BlockSpec((2, bn, a.shape[2]), lambda i: (0, i, 0)) for a in aggs]
        + [
            pl.BlockSpec((d, w1.shape[1]), lambda i: (0, 0)),
            pl.BlockSpec((1, w1.shape[1]), lambda i: (0, 0)),
            pl.BlockSpec((w1.shape[1], out_c), lambda i: (0, 0)),
            pl.BlockSpec((1, out_c), lambda i: (0, 0)),
        ],
        out_specs=pl.BlockSpec((bn, out_c), lambda i: (i, 0)),
        out_shape=jax.ShapeDtypeStruct((n, out_c), jnp.float32),
    )(h, *aggs, w1, b1, w2, b2)


def kernel(x, edge_index, params):
    n, d_in = x.shape
    e = edge_index.shape[1]
    epw = pl.cdiv(e, NW * 4 * K) * 4 * K
    n_chunks = epw // K
    e_pad = NW * epw

    src = edge_index[0]
    dst = edge_index[1]
    npad_e = e_pad - e
    pad_iota = jnp.arange(npad_e, dtype=jnp.int32)
    src_p = jnp.concatenate(
        [src, pad_iota % n]).reshape(NW, n_chunks, K)
    dst_p = jnp.concatenate(
        [dst, n + pad_iota % (N_PAD - n)]).reshape(NW, n_chunks, K)

    h = x
    for (w1, b1, w2, b2) in params:
        d = h.shape[1]
        dc = 64 if d > 64 else d
        parts = []
        for c0 in range(0, d, dc):
            hcol = h[:, c0:c0 + dc] if d > dc else h
            parts.append(_make_sc_aggregate(n_chunks, dc)(hcol, src_p, dst_p))
        h = _mlp_apply(h, parts,
                       w1, b1.reshape(1, -1), w2, b2.reshape(1, -1))
    return h


# speedup vs baseline: 3.2412x; 1.0647x over previous
import functools

import jax
import jax.numpy as jnp
from jax import lax
from jax.experimental import pallas as pl
from jax.experimental.pallas import tpu as pltpu
from jax.experimental.pallas import tpu_sc as plsc

N_NODES = 10000
N_PAD = 10240
NW = 32
K = 128
ROWS_PER_TILE = N_PAD // 16
ZCH = ROWS_PER_TILE // K


def _make_sc_aggregate(n_chunks: int, d: int):
    mesh = plsc.VectorSubcoreMesh(
        core_axis_name="c", subcore_axis_name="s", num_cores=2, num_subcores=16)

    @functools.partial(
        pl.kernel,
        mesh=mesh,
        out_type=jax.ShapeDtypeStruct((2, N_PAD, d), jnp.float32),
        scratch_types=[
            pltpu.VMEM((n_chunks, K), jnp.int32),
            pltpu.VMEM((n_chunks, K), jnp.int32),
            [pltpu.VMEM((K, d), jnp.float32)] * 4,
            pltpu.VMEM((K, d), jnp.float32),
            pltpu.VMEM_SHARED((N_PAD, d), jnp.float32),
            [pltpu.SemaphoreType.DMA] * 4,
            [pltpu.SemaphoreType.DMA] * 4,
        ],
        compiler_params=pltpu.CompilerParams(use_tc_tiling_on_sc=False),
    )
    def agg_kernel(h_hbm, src_hbm, dst_hbm, out_hbm,
                   src_v, dst_v, rows, zbuf, acc, gsem, ssem):
        c = lax.axis_index("c")
        s = lax.axis_index("s")
        wid = c * 16 + s

        zv = jnp.zeros((16,), jnp.float32)

        def zrow(i, _):
            for j in range(d // 16):
                zbuf[i, pl.ds(j * 16, 16)] = zv
            return 0

        lax.fori_loop(0, K, zrow, 0)
        for z in range(ZCH):
            pltpu.sync_copy(zbuf, acc.at[pl.ds(s * ROWS_PER_TILE + z * K, K)])
        plsc.subcore_barrier()

        pltpu.sync_copy(src_hbm.at[wid], src_v)
        pltpu.sync_copy(dst_hbm.at[wid], dst_v)

        nb = 4
        for b in range(nb):
            pltpu.async_copy(h_hbm.at[src_v.at[b]], rows[b], gsem[b])

        n_groups = n_chunks // nb

        def group(g, _):
            for b in range(nb):
                j = g * nb + b
                pltpu.make_async_copy(h_hbm.at[src_v.at[j]], rows[b], gsem[b]).wait()
                pltpu.async_copy(rows[b], acc.at[dst_v.at[j]], ssem[b], add=True)
            for b in range(nb):
                j = g * nb + b
                pltpu.make_async_copy(rows[b], acc.at[dst_v.at[j]], ssem[b]).wait()
                pltpu.async_copy(h_hbm.at[src_v.at[j + nb]], rows[b], gsem[b])
            return 0

        lax.fori_loop(0, n_groups - 1, group, 0)

        t0 = (n_groups - 1) * nb
        for b in range(nb):
            j = t0 + b
            pltpu.make_async_copy(h_hbm.at[src_v.at[j]], rows[b], gsem[b]).wait()
            pltpu.async_copy(rows[b], acc.at[dst_v.at[j]], ssem[b], add=True)
        for b in range(nb):
            j = t0 + b
            pltpu.make_async_copy(rows[b], acc.at[dst_v.at[j]], ssem[b]).wait()

        plsc.subcore_barrier()
        pltpu.sync_copy(
            acc.at[pl.ds(s * ROWS_PER_TILE, ROWS_PER_TILE)],
            out_hbm.at[c, pl.ds(s * ROWS_PER_TILE, ROWS_PER_TILE)],
        )

    return agg_kernel


def _mlp_apply(h, aggs, w1, b1, w2, b2):
    n, d = h.shape
    out_c = w2.shape[1]
    bn = 512
    grid = (pl.cdiv(n, bn),)
    n_parts = len(aggs)

    def body(h_ref, *refs):
        agg_refs = refs[:n_parts]
        w1_ref, b1_ref, w2_ref, b2_ref, o_ref = refs[n_parts:]
        parts = [ar[0] + ar[1] for ar in agg_refs]
        agg = jnp.concatenate(parts, axis=-1) if n_parts > 1 else parts[0]
        z = h_ref[...] + agg
        y = jnp.maximum(
            jnp.dot(z, w1_ref[...], preferred_element_type=jnp.float32,
                    precision=lax.Precision.HIGHEST) + b1_ref[...], 0.0)
        o = jnp.dot(y, w2_ref[...], preferred_element_type=jnp.float32,
                    precision=lax.Precision.HIGHEST) + b2_ref[...]
        o_ref[...] = jnp.maximum(o, 0.0)

    return pl.pallas_call(
        body,
        grid=grid,
        in_specs=[pl.BlockSpec((bn, d), lambda i: (i, 0))]
        + [pl.BlockSpec((2, bn, a.shape[2]), lambda i: (0, i, 0)) for a in aggs]
        + [
            pl.BlockSpec((d, w1.shape[1]), lambda i: (0, 0)),
            pl.BlockSpec((1, w1.shape[1]), lambda i: (0, 0)),
            pl.BlockSpec((w1.shape[1], out_c), lambda i: (0, 0)),
            pl.BlockSpec((1, out_c), lambda i: (0, 0)),
        ],
        out_specs=pl.BlockSpec((bn, out_c), lambda i: (i, 0)),
        out_shape=jax.ShapeDtypeStruct((n, out_c), jnp.float32),
    )(h, *aggs, w1, b1, w2, b2)


def kernel(x, edge_index, params):
    n, d_in = x.shape
    e = edge_index.shape[1]
    epw = pl.cdiv(e, NW * 4 * K) * 4 * K
    n_chunks = epw // K
    e_pad = NW * epw

    src = edge_index[0]
    dst = edge_index[1]
    npad_e = e_pad - e
    pad_iota = jnp.arange(npad_e, dtype=jnp.int32)
    src_p = jnp.concatenate(
        [src, pad_iota % n]).reshape(NW, n_chunks, K)
    dst_p = jnp.concatenate(
        [dst, n + pad_iota % (N_PAD - n)]).reshape(NW, n_chunks, K)

    h = x
    for (w1, b1, w2, b2) in params:
        d = h.shape[1]
        dc = 64 if d > 64 else d
        parts = []
        for c0 in range(0, d, dc):
            hcol = h[:, c0:c0 + dc] if d > dc else h
            parts.append(_make_sc_aggregate(n_chunks, dc)(hcol, src_p, dst_p))
        h = _mlp_apply(h, parts,
                       w1, b1.reshape(1, -1), w2, b2.reshape(1, -1))
    return h
